# baseline (speedup 1.0000x reference)
import functools

import jax
import jax.numpy as jnp
from jax import lax
from jax.experimental import pallas as pl
from jax.experimental.pallas import tpu as pltpu
from jax.experimental.pallas import tpu_sc as plsc

B = 128
V = 100000
RB = 8
NR = B // RB
RPW = 4


def _sc_gather_kernel(pred_hbm, tgt_hbm, out_hbm, tgt_v, blk_v, val_v, sem):
    core = lax.axis_index("c")
    sub = lax.axis_index("s")
    wid = sub * 2 + core
    lanes = lax.iota(jnp.int32, 16)

    tbase = pl.multiple_of((wid // 4) * 16, 8)
    pltpu.sync_copy(tgt_hbm.at[pl.ds(tbase, 16)], tgt_v)
    t = tgt_v[...]

    rb = pl.multiple_of((wid // 2) * 8, 8)
    copies = []
    tj = []
    for j in range(RPW):
        lane = (wid % 4) * 4 + j
        t_j = jnp.sum(t * (lanes == lane).astype(jnp.int32))
        tj.append(t_j)
        cb = pl.multiple_of(jnp.bitwise_and(t_j, -128), 128)
        copies.append(pltpu.async_copy(
            pred_hbm.at[pl.ds(rb, 8), pl.ds(cb, 128)], blk_v.at[j], sem))
    for c in copies:
        c.wait()
    for j in range(RPW):
        rit = (wid % 2) * 4 + j
        off = jnp.bitwise_and(tj[j], 127)
        x16 = blk_v[j, rit, pl.ds(jnp.bitwise_and(off, -16), 16)]
        sel = (lanes == jnp.bitwise_and(off, 15)).astype(jnp.float32)
        v_j = jnp.sum(x16 * sel)
        val_v[j, :] = jnp.full((16,), v_j, dtype=jnp.float32)
    pltpu.sync_copy(val_v, out_hbm.at[pl.ds(wid * RPW, RPW)])


def _sc_gather(predicted, targets):
    mesh = plsc.VectorSubcoreMesh(core_axis_name="c", subcore_axis_name="s")
    kfn = functools.partial(
        pl.kernel,
        mesh=mesh,
        compiler_params=pltpu.CompilerParams(needs_layout_passes=False),
        out_type=jax.ShapeDtypeStruct((B, 16), jnp.float32),
        scratch_types=[
            pltpu.VMEM((16,), jnp.int32),
            pltpu.VMEM((RPW, 8, 128), jnp.float32),
            pltpu.VMEM((RPW, 16), jnp.float32),
            pltpu.SemaphoreType.DMA,
        ],
    )(_sc_gather_kernel)
    return kfn(predicted, targets)


def _tc_count_kernel(pred_ref, tgt_ref, v_ref, out_ref, acc_ref):
    c = pl.program_id(0)
    x = pred_ref[...]
    v = v_ref[:, 0:1]
    t = tgt_ref[...]
    col = lax.broadcasted_iota(jnp.int32, (RB, V), 1)
    beats = (x > v) | ((x == v) & (col < t))
    cnt = jnp.sum(beats.astype(jnp.float32), axis=1, keepdims=True)
    acc_ref[pl.ds(c * RB, RB), :] = cnt

    @pl.when(c == NR - 1)
    def _fini():
        miss = (acc_ref[...] >= 29.5).astype(jnp.float32)
        out_ref[...] = jnp.sum(miss, axis=0, keepdims=True) * (1.0 / B)


def _tc_count(predicted, targets2d, v2d):
    return pl.pallas_call(
        _tc_count_kernel,
        grid=(NR,),
        in_specs=[
            pl.BlockSpec((RB, V), lambda c: (c, 0)),
            pl.BlockSpec((RB, 1), lambda c: (c, 0)),
            pl.BlockSpec((RB, 16), lambda c: (c, 0)),
        ],
        out_specs=pl.BlockSpec((1, 1), lambda c: (0, 0)),
        out_shape=jax.ShapeDtypeStruct((1, 1), jnp.float32),
        scratch_shapes=[pltpu.VMEM((B, 1), jnp.float32)],
    )(predicted, targets2d, v2d)


def kernel(predicted, targets):
    v2d = _sc_gather(predicted, targets)
    out = _tc_count(predicted, targets.reshape(B, 1), v2d)
    return out[0, 0]


# speedup vs baseline: 1.0470x; 1.0470x over previous
import functools

import jax
import jax.numpy as jnp
from jax import lax
from jax.experimental import pallas as pl
from jax.experimental.pallas import tpu as pltpu
from jax.experimental.pallas import tpu_sc as plsc

B = 128
V = 100000
RB = 16
NR = B // RB
RPW = 4


def _sc_gather_kernel(pred_hbm, tgt_hbm, out_hbm, tgt_v, blk_v, val_v, sem):
    core = lax.axis_index("c")
    sub = lax.axis_index("s")
    wid = sub * 2 + core
    lanes = lax.iota(jnp.int32, 16)

    tbase = pl.multiple_of((wid // 4) * 16, 8)
    pltpu.sync_copy(tgt_hbm.at[pl.ds(tbase, 16)], tgt_v)
    t = tgt_v[...]

    rb = pl.multiple_of((wid // 2) * 8, 8)
    copies = []
    tj = []
    for j in range(RPW):
        lane = (wid % 4) * 4 + j
        t_j = jnp.sum(t * (lanes == lane).astype(jnp.int32))
        tj.append(t_j)
        cb = pl.multiple_of(jnp.bitwise_and(t_j, -128), 128)
        copies.append(pltpu.async_copy(
            pred_hbm.at[pl.ds(rb, 8), pl.ds(cb, 128)], blk_v.at[j], sem))
    for c in copies:
        c.wait()
    for j in range(RPW):
        rit = (wid % 2) * 4 + j
        off = jnp.bitwise_and(tj[j], 127)
        x16 = blk_v[j, rit, pl.ds(jnp.bitwise_and(off, -16), 16)]
        sel = (lanes == jnp.bitwise_and(off, 15)).astype(jnp.float32)
        v_j = jnp.sum(x16 * sel)
        val_v[j, :] = jnp.full((16,), v_j, dtype=jnp.float32)
    pltpu.sync_copy(val_v, out_hbm.at[pl.ds(wid * RPW, RPW)])


def _sc_gather(predicted, targets):
    mesh = plsc.VectorSubcoreMesh(core_axis_name="c", subcore_axis_name="s")
    kfn = functools.partial(
        pl.kernel,
        mesh=mesh,
        compiler_params=pltpu.CompilerParams(needs_layout_passes=False),
        out_type=jax.ShapeDtypeStruct((B, 16), jnp.float32),
        scratch_types=[
            pltpu.VMEM((16,), jnp.int32),
            pltpu.VMEM((RPW, 8, 128), jnp.float32),
            pltpu.VMEM((RPW, 16), jnp.float32),
            pltpu.SemaphoreType.DMA,
        ],
    )(_sc_gather_kernel)
    return kfn(predicted, targets)


def _tc_count_kernel(pred_ref, tgt_ref, v_ref, out_ref, acc_ref):
    c = pl.program_id(0)
    x = pred_ref[...]
    v = v_ref[:, 0:1]
    t = tgt_ref[...]
    col = lax.broadcasted_iota(jnp.int32, (RB, V), 1)
    beats = (x > v) | ((x == v) & (col < t))
    cnt = jnp.sum(beats.astype(jnp.float32), axis=1, keepdims=True)
    acc_ref[pl.ds(c * RB, RB), :] = cnt

    @pl.when(c == NR - 1)
    def _fini():
        miss = (acc_ref[...] >= 29.5).astype(jnp.float32)
        out_ref[...] = jnp.sum(miss, axis=0, keepdims=True) * (1.0 / B)


def _tc_count(predicted, targets2d, v2d):
    return pl.pallas_call(
        _tc_count_kernel,
        grid=(NR,),
        in_specs=[
            pl.BlockSpec((RB, V), lambda c: (c, 0)),
            pl.BlockSpec((RB, 1), lambda c: (c, 0)),
            pl.BlockSpec((RB, 16), lambda c: (c, 0)),
        ],
        out_specs=pl.BlockSpec((1, 1), lambda c: (0, 0)),
        out_shape=jax.ShapeDtypeStruct((1, 1), jnp.float32),
        scratch_shapes=[pltpu.VMEM((B, 1), jnp.float32)],
    )(predicted, targets2d, v2d)


def kernel(predicted, targets):
    v2d = _sc_gather(predicted, targets)
    out = _tc_count(predicted, targets.reshape(B, 1), v2d)
    return out[0, 0]


# speedup vs baseline: 1.0572x; 1.0098x over previous
import functools

import jax
import jax.numpy as jnp
from jax import lax
from jax.experimental import pallas as pl
from jax.experimental.pallas import tpu as pltpu
from jax.experimental.pallas import tpu_sc as plsc

B = 128
V = 100000
RB = 32
NR = B // RB
RPW = 4


def _sc_gather_kernel(pred_hbm, tgt_hbm, out_hbm, tgt_v, blk_v, val_v, sem):
    core = lax.axis_index("c")
    sub = lax.axis_index("s")
    wid = sub * 2 + core
    lanes = lax.iota(jnp.int32, 16)

    tbase = pl.multiple_of((wid // 4) * 16, 8)
    pltpu.sync_copy(tgt_hbm.at[pl.ds(tbase, 16)], tgt_v)
    t = tgt_v[...]

    rb = pl.multiple_of((wid // 2) * 8, 8)
    copies = []
    tj = []
    for j in range(RPW):
        lane = (wid % 4) * 4 + j
        t_j = jnp.sum(t * (lanes == lane).astype(jnp.int32))
        tj.append(t_j)
        cb = pl.multiple_of(jnp.bitwise_and(t_j, -128), 128)
        copies.append(pltpu.async_copy(
            pred_hbm.at[pl.ds(rb, 8), pl.ds(cb, 128)], blk_v.at[j], sem))
    for c in copies:
        c.wait()
    for j in range(RPW):
        rit = (wid % 2) * 4 + j
        off = jnp.bitwise_and(tj[j], 127)
        x16 = blk_v[j, rit, pl.ds(jnp.bitwise_and(off, -16), 16)]
        sel = (lanes == jnp.bitwise_and(off, 15)).astype(jnp.float32)
        v_j = jnp.sum(x16 * sel)
        val_v[j, :] = jnp.full((16,), v_j, dtype=jnp.float32)
    pltpu.sync_copy(val_v, out_hbm.at[pl.ds(wid * RPW, RPW)])


def _sc_gather(predicted, targets):
    mesh = plsc.VectorSubcoreMesh(core_axis_name="c", subcore_axis_name="s")
    kfn = functools.partial(
        pl.kernel,
        mesh=mesh,
        compiler_params=pltpu.CompilerParams(needs_layout_passes=False),
        out_type=jax.ShapeDtypeStruct((B, 16), jnp.float32),
        scratch_types=[
            pltpu.VMEM((16,), jnp.int32),
            pltpu.VMEM((RPW, 8, 128), jnp.float32),
            pltpu.VMEM((RPW, 16), jnp.float32),
            pltpu.SemaphoreType.DMA,
        ],
    )(_sc_gather_kernel)
    return kfn(predicted, targets)


def _tc_count_kernel(pred_ref, tgt_ref, v_ref, out_ref, acc_ref):
    c = pl.program_id(0)
    x = pred_ref[...]
    v = v_ref[:, 0:1]
    t = tgt_ref[...]
    col = lax.broadcasted_iota(jnp.int32, (RB, V), 1)
    beats = (x > v) | ((x == v) & (col < t))
    cnt = jnp.sum(beats.astype(jnp.float32), axis=1, keepdims=True)
    acc_ref[pl.ds(c * RB, RB), :] = cnt

    @pl.when(c == NR - 1)
    def _fini():
        miss = (acc_ref[...] >= 29.5).astype(jnp.float32)
        out_ref[...] = jnp.sum(miss, axis=0, keepdims=True) * (1.0 / B)


def _tc_count(predicted, targets2d, v2d):
    return pl.pallas_call(
        _tc_count_kernel,
        grid=(NR,),
        in_specs=[
            pl.BlockSpec((RB, V), lambda c: (c, 0)),
            pl.BlockSpec((RB, 1), lambda c: (c, 0)),
            pl.BlockSpec((RB, 16), lambda c: (c, 0)),
        ],
        out_specs=pl.BlockSpec((1, 1), lambda c: (0, 0)),
        out_shape=jax.ShapeDtypeStruct((1, 1), jnp.float32),
        scratch_shapes=[pltpu.VMEM((B, 1), jnp.float32)],
    )(predicted, targets2d, v2d)


def kernel(predicted, targets):
    v2d = _sc_gather(predicted, targets)
    out = _tc_count(predicted, targets.reshape(B, 1), v2d)
    return out[0, 0]
